# baseline (speedup 1.0000x reference)
import functools

import jax
import jax.numpy as jnp
from jax import lax
from jax.experimental import pallas as pl
from jax.experimental.pallas import tpu as pltpu
from jax.experimental.pallas import tpu_sc as plsc

_B = 8
_L = 63 * 63
_CBIG = 0.949
_TLO = 0.8999
_FBIG = _CBIG / (1.0 - _CBIG) * 1e-12
_FMID = _TLO / (1.0 - _TLO) * 1e-12


def _sigmoid(z):
    t = jnp.clip(-z, -87.0, 88.0)
    magic = jnp.float32(12582912.0)
    nf = t * jnp.float32(1.4426950408889634) + magic
    n = nf - magic
    ni = n.astype(jnp.int32)
    r = (t - n * jnp.float32(0.693359375)) - n * jnp.float32(-2.12194440e-4)
    p = jnp.float32(1.0 / 720.0)
    for c in (1.0 / 120.0, 1.0 / 24.0, 1.0 / 6.0, 0.5, 1.0, 1.0):
        p = p * r + jnp.float32(c)
    scale = lax.bitcast_convert_type((ni + 127) << 23, jnp.float32)
    d = 1.0 + p * scale
    y = 1.0 / d
    y = y * (2.0 - d * y)
    y = y * (2.0 - d * y)
    return y


def _sc_graph_mean(x2d, wpack):
    mesh = plsc.VectorSubcoreMesh(core_axis_name="c", subcore_axis_name="s")

    @functools.partial(
        pl.kernel,
        out_type=jax.ShapeDtypeStruct((_B, 16), jnp.float32),
        mesh=mesh,
        compiler_params=pltpu.CompilerParams(needs_layout_passes=False),
        scratch_types=[
            pltpu.VMEM((4096,), jnp.float32),
            pltpu.VMEM((80,), jnp.float32),
            pltpu.VMEM((16,), jnp.float32),
            pltpu.VMEM((4096,), jnp.float32),
            pltpu.VMEM((4096,), jnp.float32),
        ],
    )
    def k(x_hbm, w_hbm, out_hbm, x_v, w_v, o_v, f_st, nz_st):
        cid = lax.axis_index("c")
        sid = lax.axis_index("s")
        wid = sid * 2 + cid

        @pl.when(wid < _B)
        def _():
            img = wid
            pltpu.sync_copy(x_hbm.at[img], x_v)
            pltpu.sync_copy(w_hbm, w_v)
            iota = lax.iota(jnp.int32, 16)
            w0 = w_v[pl.ds(0, 16)]
            w1 = w_v[pl.ds(16, 16)]
            w2 = w_v[pl.ds(32, 16)]
            w3 = w_v[pl.ds(48, 16)]
            bb = w_v[pl.ds(64, 16)]

            def row_body(r, carry):
                s_f, s_b, n_b, n_m = carry
                base = r * 64
                for kk in range(4):
                    j = iota + (kk * 16)
                    ia = j + base
                    ic = ia + 64
                    a = plsc.load_gather(x_v, [ia])
                    bq = plsc.load_gather(x_v, [ia + 1])
                    c = plsc.load_gather(x_v, [ic])
                    dq = plsc.load_gather(x_v, [jnp.minimum(ic + 1, 4095)])
                    z = a * w0 + bq * w1 + c * w2 + dq * w3 + bb
                    f = _sigmoid(z)
                    valid = j < 63
                    f = jnp.where(valid, f, 0.0)
                    is_b = valid & (f >= _FBIG)
                    is_m = valid & (f >= _FMID) & (f < _FBIG)
                    s_f = s_f + f
                    s_b = s_b + jnp.where(is_b, f, 0.0)
                    n_b = n_b + jnp.where(is_b, 1.0, 0.0)
                    n_m = n_m + jnp.where(is_m, 1.0, 0.0)
                    f_st[pl.ds(base + kk * 16, 16)] = f
                return (s_f, s_b, n_b, n_m)

            zv = jnp.zeros((16,), jnp.float32)
            s_f, s_b, n_b, n_m = lax.fori_loop(0, 63, row_body, (zv, zv, zv, zv))
            tot = jnp.sum(s_f)
            tot_b = jnp.sum(s_b)
            nb = jnp.sum(n_b)
            nm = jnp.sum(n_m)
            inv_l = jnp.float32(1.0 / _L)

            def fast_fn(_):
                return jnp.where(nb >= 2.0, (tot + tot_b) * inv_l, tot * inv_l)

            def exact_fn(_):
                nch = 252
                t9 = jnp.float32(0.9)

                def nz_body(ch, c):
                    fc = f_st[pl.ds(ch * 16, 16)]
                    dd = fc + 1e-12
                    y = 1.0 / dd
                    y = y * (2.0 - dd * y)
                    y = y * (2.0 - dd * y)
                    nz_st[pl.ds(ch * 16, 16)] = fc * y
                    return c

                lax.fori_loop(0, nch, nz_body, 0)

                def i_body(ich, tvec):
                    fv = f_st[pl.ds(ich * 16, 16)]
                    nv = nz_st[pl.ds(ich * 16, 16)]
                    deg_v = jnp.zeros((16,), jnp.float32)
                    agg_v = jnp.zeros((16,), jnp.float32)
                    for l in range(16):
                        nz_i = jnp.reshape(lax.slice(nv, (l,), (l + 1,)), ())
                        f_i = jnp.reshape(lax.slice(fv, (l,), (l + 1,)), ())
                        spl = jnp.full((16,), nz_i, jnp.float32)

                        def j_body(jch, c):
                            cnt, s = c
                            nzj = nz_st[pl.ds(jch * 16, 16)]
                            fj = f_st[pl.ds(jch * 16, 16)]
                            e = (spl * nzj) >= t9
                            return (cnt + jnp.where(e, 1.0, 0.0),
                                    s + jnp.where(e, fj, 0.0))

                        zz = jnp.zeros((16,), jnp.float32)
                        cnt, s = lax.fori_loop(0, nch, j_body, (zz, zz))
                        se = jnp.where(nz_i * nz_i >= t9, 1.0, 0.0)
                        deg = jnp.sum(cnt) - se
                        agg = jnp.sum(s) - se * f_i
                        deg_v = jnp.where(iota == l, deg, deg_v)
                        agg_v = jnp.where(iota == l, agg, agg_v)
                    dd = jnp.where(deg_v > 0, deg_v, 1.0)
                    y = 1.0 / dd
                    y = y * (2.0 - dd * y)
                    y = y * (2.0 - dd * y)
                    contrib = fv + jnp.where(deg_v > 0, agg_v * y, 0.0)
                    return tvec + contrib

                tvec = lax.fori_loop(0, nch, i_body, jnp.zeros((16,), jnp.float32))
                return jnp.sum(tvec) * inv_l

            res = lax.cond(nm > 0, exact_fn, fast_fn, 0)
            o_v[...] = jnp.where(iota == 0, res, jnp.where(iota == 1, nm, 0.0))
            pltpu.sync_copy(o_v, out_hbm.at[img])

    return k(x2d, wpack)


def kernel(x, W, b):
    x2d = x.reshape(_B, 64 * 64)
    wpack = jnp.repeat(
        jnp.concatenate([W.reshape(-1), b.reshape(-1)]).astype(jnp.float32), 16
    )
    stats = _sc_graph_mean(x2d, wpack)
    return stats[:, :1]


# speedup vs baseline: 1.2958x; 1.2958x over previous
import functools

import jax
import jax.numpy as jnp
from jax import lax
from jax.experimental import pallas as pl
from jax.experimental.pallas import tpu as pltpu
from jax.experimental.pallas import tpu_sc as plsc

_B = 8
_L = 63 * 63
_CBIG = 0.949
_TLO = 0.8999
_FBIG = _CBIG / (1.0 - _CBIG) * 1e-12
_FMID = _TLO / (1.0 - _TLO) * 1e-12


def _sigmoid(z):
    t = jnp.clip(-z, -87.0, 88.0)
    magic = jnp.float32(12582912.0)
    nf = t * jnp.float32(1.4426950408889634) + magic
    n = nf - magic
    ni = n.astype(jnp.int32)
    r = (t - n * jnp.float32(0.693359375)) - n * jnp.float32(-2.12194440e-4)
    p = jnp.float32(1.0 / 720.0)
    for c in (1.0 / 120.0, 1.0 / 24.0, 1.0 / 6.0, 0.5, 1.0, 1.0):
        p = p * r + jnp.float32(c)
    scale = lax.bitcast_convert_type((ni + 127) << 23, jnp.float32)
    d = 1.0 + p * scale
    y = 1.0 / d
    y = y * (2.0 - d * y)
    y = y * (2.0 - d * y)
    return y


def _sc_graph_mean(xflat, wpack):
    mesh = plsc.VectorSubcoreMesh(core_axis_name="c", subcore_axis_name="s")

    @functools.partial(
        pl.kernel,
        out_type=jax.ShapeDtypeStruct((_B, 16), jnp.float32),
        mesh=mesh,
        compiler_params=pltpu.CompilerParams(needs_layout_passes=False),
        scratch_types=[
            pltpu.VMEM((1088,), jnp.float32),
            pltpu.VMEM((80,), jnp.float32),
            pltpu.VMEM((16,), jnp.float32),
            pltpu.VMEM((1024,), jnp.float32),
            pltpu.VMEM((64,), jnp.float32),
            pltpu.VMEM((4096,), jnp.float32),
            pltpu.VMEM((4096,), jnp.float32),
            pltpu.VMEM_SHARED((256,), jnp.float32),
            pltpu.VMEM_SHARED((16384,), jnp.float32),
        ],
    )
    def k(x_hbm, w_hbm, out_hbm, x_v, w_v, o_v, fseg, p4, f_full, nz_st,
          shp, shf):
        cid = lax.axis_index("c")
        sid = lax.axis_index("s")
        il = sid // 4
        g = sid % 4
        img = cid * 4 + il
        off = img * 4096 + g * 1024

        pltpu.sync_copy(x_hbm.at[pl.ds(off, 1088)], x_v)
        pltpu.sync_copy(w_hbm, w_v)
        iota = lax.iota(jnp.int32, 16)
        w0 = w_v[pl.ds(0, 16)]
        w1 = w_v[pl.ds(16, 16)]
        w2 = w_v[pl.ds(32, 16)]
        w3 = w_v[pl.ds(48, 16)]
        bb = w_v[pl.ds(64, 16)]
        last_group = g == 3

        def row_body(r, carry):
            s_f, s_b, n_b, n_m = carry
            base = r * 64
            row_ok = jnp.logical_or(jnp.logical_not(last_group), r < 15)
            for kk in range(4):
                jcol = iota + (kk * 16)
                p0 = base + kk * 16
                a = x_v[pl.ds(p0, 16)]
                c = x_v[pl.ds(p0 + 64, 16)]
                bq = plsc.load_gather(x_v, [jcol + (base + 1)])
                dq = plsc.load_gather(
                    x_v, [jnp.minimum(jcol + (base + 65), 1087)])
                z = a * w0 + bq * w1 + c * w2 + dq * w3 + bb
                f = _sigmoid(z)
                f = jnp.where((jcol < 63) & row_ok, f, 0.0)
                is_b = f >= _FBIG
                is_m = (f >= _FMID) & (f < _FBIG)
                s_f = s_f + f
                s_b = s_b + jnp.where(is_b, f, 0.0)
                n_b = n_b + jnp.where(is_b, 1.0, 0.0)
                n_m = n_m + jnp.where(is_m, 1.0, 0.0)
                fseg[pl.ds(p0, 16)] = f
            return (s_f, s_b, n_b, n_m)

        zv = jnp.zeros((16,), jnp.float32)
        s_f, s_b, n_b, n_m = lax.fori_loop(0, 16, row_body, (zv, zv, zv, zv))
        part = jnp.where(iota == 0, jnp.sum(s_f),
                         jnp.where(iota == 1, jnp.sum(s_b),
                                   jnp.where(iota == 2, jnp.sum(n_b),
                                             jnp.where(iota == 3,
                                                       jnp.sum(n_m), 0.0))))
        o_v[...] = part
        pltpu.sync_copy(o_v, shp.at[pl.ds(sid * 16, 16)])
        pltpu.sync_copy(fseg, shf.at[pl.ds(il * 4096 + g * 1024, 1024)])
        plsc.subcore_barrier()

        @pl.when(g == 3)
        def _():
            for kq in range(4):
                pltpu.sync_copy(shp.at[pl.ds((sid - 3 + kq) * 16, 16)],
                                p4.at[pl.ds(kq * 16, 16)])
            vsum = (p4[pl.ds(0, 16)] + p4[pl.ds(16, 16)]
                    + p4[pl.ds(32, 16)] + p4[pl.ds(48, 16)])

            def lane(v, i):
                return jnp.reshape(lax.slice(v, (i,), (i + 1,)), ())

            tot = lane(vsum, 0)
            tot_b = lane(vsum, 1)
            nb = lane(vsum, 2)
            nm = lane(vsum, 3)
            inv_l = jnp.float32(1.0 / _L)

            def fast_fn(_):
                return jnp.where(nb >= 2.0, (tot + tot_b) * inv_l,
                                 tot * inv_l)

            def exact_fn(_):
                nch = 252
                t9 = jnp.float32(0.9)
                pltpu.sync_copy(shf.at[pl.ds(il * 4096, 4096)], f_full)

                def nz_body(ch, cc):
                    fc = f_full[pl.ds(ch * 16, 16)]
                    dd = fc + 1e-12
                    y = 1.0 / dd
                    y = y * (2.0 - dd * y)
                    y = y * (2.0 - dd * y)
                    nz_st[pl.ds(ch * 16, 16)] = fc * y
                    return cc

                lax.fori_loop(0, nch, nz_body, 0)

                def i_body(ich, tvec):
                    fv = f_full[pl.ds(ich * 16, 16)]
                    nv = nz_st[pl.ds(ich * 16, 16)]
                    deg_v = jnp.zeros((16,), jnp.float32)
                    agg_v = jnp.zeros((16,), jnp.float32)
                    for l in range(16):
                        nz_i = lane(nv, l)
                        f_i = lane(fv, l)
                        spl = jnp.full((16,), nz_i, jnp.float32)

                        def j_body(jch, c):
                            cnt, s = c
                            nzj = nz_st[pl.ds(jch * 16, 16)]
                            fj = f_full[pl.ds(jch * 16, 16)]
                            e = (spl * nzj) >= t9
                            return (cnt + jnp.where(e, 1.0, 0.0),
                                    s + jnp.where(e, fj, 0.0))

                        zz = jnp.zeros((16,), jnp.float32)
                        cnt, s = lax.fori_loop(0, nch, j_body, (zz, zz))
                        se = jnp.where(nz_i * nz_i >= t9, 1.0, 0.0)
                        deg = jnp.sum(cnt) - se
                        agg = jnp.sum(s) - se * f_i
                        deg_v = jnp.where(iota == l, deg, deg_v)
                        agg_v = jnp.where(iota == l, agg, agg_v)
                    dd = jnp.where(deg_v > 0, deg_v, 1.0)
                    y = 1.0 / dd
                    y = y * (2.0 - dd * y)
                    y = y * (2.0 - dd * y)
                    contrib = fv + jnp.where(deg_v > 0, agg_v * y, 0.0)
                    return tvec + contrib

                tvec = lax.fori_loop(0, nch, i_body,
                                     jnp.zeros((16,), jnp.float32))
                return jnp.sum(tvec) * inv_l

            res = lax.cond(nm > 0, exact_fn, fast_fn, 0)
            o_v[...] = jnp.where(iota == 0, res,
                                 jnp.where(iota == 1, nm, 0.0))
            pltpu.sync_copy(o_v, out_hbm.at[img])

    return k(xflat, wpack)


def kernel(x, W, b):
    xflat = jnp.concatenate(
        [x.reshape(-1), jnp.zeros((64,), jnp.float32)])
    wpack = jnp.repeat(
        jnp.concatenate([W.reshape(-1), b.reshape(-1)]).astype(jnp.float32), 16
    )
    stats = _sc_graph_mean(xflat, wpack)
    return stats[:, :1]


# speedup vs baseline: 1.4052x; 1.0844x over previous
import functools

import jax
import jax.numpy as jnp
from jax import lax
from jax.experimental import pallas as pl
from jax.experimental.pallas import tpu as pltpu
from jax.experimental.pallas import tpu_sc as plsc

_B = 8
_L = 63 * 63
_CBIG = 0.949
_TLO = 0.8999
_FBIG = _CBIG / (1.0 - _CBIG) * 1e-12
_FMID = _TLO / (1.0 - _TLO) * 1e-12


def _sigmoid(z):
    t = jnp.clip(-z, -87.0, 88.0)
    magic = jnp.float32(12582912.0)
    nf = t * jnp.float32(1.4426950408889634) + magic
    n = nf - magic
    ni = n.astype(jnp.int32)
    r = (t - n * jnp.float32(0.693359375)) - n * jnp.float32(-2.12194440e-4)
    p = jnp.float32(1.0 / 720.0)
    for c in (1.0 / 120.0, 1.0 / 24.0, 1.0 / 6.0, 0.5, 1.0, 1.0):
        p = p * r + jnp.float32(c)
    scale = lax.bitcast_convert_type((ni + 127) << 23, jnp.float32)
    d = 1.0 + p * scale
    y = 1.0 / d
    y = y * (2.0 - d * y)
    y = y * (2.0 - d * y)
    return y


def _sc_graph_mean(xflat, wpack):
    mesh = plsc.VectorSubcoreMesh(core_axis_name="c", subcore_axis_name="s")

    @functools.partial(
        pl.kernel,
        out_type=jax.ShapeDtypeStruct((_B, 16), jnp.float32),
        mesh=mesh,
        compiler_params=pltpu.CompilerParams(needs_layout_passes=False),
        scratch_types=[
            pltpu.VMEM((1088,), jnp.float32),
            pltpu.VMEM((80,), jnp.float32),
            pltpu.VMEM((16,), jnp.float32),
            pltpu.VMEM((64,), jnp.float32),
            pltpu.VMEM((4096,), jnp.float32),
            pltpu.VMEM((4096,), jnp.float32),
            pltpu.VMEM((4096,), jnp.float32),
            pltpu.VMEM_SHARED((256,), jnp.float32),
        ],
    )
    def k(x_hbm, w_hbm, out_hbm, x_v, w_v, o_v, p4, x_full, f_full, nz_st,
          shp):
        cid = lax.axis_index("c")
        sid = lax.axis_index("s")
        il = sid // 4
        g = sid % 4
        img = cid * 4 + il
        off = img * 4096 + g * 1024

        @pl.when(g == 3)
        def _():
            pltpu.sync_copy(x_hbm.at[pl.ds(off, 1024)],
                            x_v.at[pl.ds(0, 1024)])

        @pl.when(g != 3)
        def _():
            pltpu.sync_copy(x_hbm.at[pl.ds(off, 1088)], x_v)

        pltpu.sync_copy(w_hbm, w_v)
        iota = lax.iota(jnp.int32, 16)
        w0 = w_v[pl.ds(0, 16)]
        w1 = w_v[pl.ds(16, 16)]
        w2 = w_v[pl.ds(32, 16)]
        w3 = w_v[pl.ds(48, 16)]
        bb = w_v[pl.ds(64, 16)]
        last_group = g == 3

        def row_body(r, carry):
            s_f, s_b, n_b, n_m = carry
            base = r * 64
            row_ok = jnp.logical_or(jnp.logical_not(last_group), r < 15)
            for kk in range(4):
                jcol = iota + (kk * 16)
                p0 = base + kk * 16
                a = x_v[pl.ds(p0, 16)]
                c = x_v[pl.ds(p0 + 64, 16)]
                bq = plsc.load_gather(x_v, [jcol + (base + 1)])
                dq = plsc.load_gather(
                    x_v, [jnp.minimum(jcol + (base + 65), 1087)])
                z = a * w0 + bq * w1 + c * w2 + dq * w3 + bb
                f = _sigmoid(z)
                f = jnp.where((jcol < 63) & row_ok, f, 0.0)
                is_b = f >= _FBIG
                is_m = (f >= _FMID) & (f < _FBIG)
                s_f = s_f + f
                s_b = s_b + jnp.where(is_b, f, 0.0)
                n_b = n_b + jnp.where(is_b, 1.0, 0.0)
                n_m = n_m + jnp.where(is_m, 1.0, 0.0)
            return (s_f, s_b, n_b, n_m)

        zv = jnp.zeros((16,), jnp.float32)
        s_f, s_b, n_b, n_m = lax.fori_loop(0, 16, row_body, (zv, zv, zv, zv))
        part = jnp.where(iota == 0, jnp.sum(s_f),
                         jnp.where(iota == 1, jnp.sum(s_b),
                                   jnp.where(iota == 2, jnp.sum(n_b),
                                             jnp.where(iota == 3,
                                                       jnp.sum(n_m), 0.0))))
        o_v[...] = part
        pltpu.sync_copy(o_v, shp.at[pl.ds(sid * 16, 16)])
        plsc.subcore_barrier()

        @pl.when(g == 3)
        def _():
            for kq in range(4):
                pltpu.sync_copy(shp.at[pl.ds((sid - 3 + kq) * 16, 16)],
                                p4.at[pl.ds(kq * 16, 16)])
            vsum = (p4[pl.ds(0, 16)] + p4[pl.ds(16, 16)]
                    + p4[pl.ds(32, 16)] + p4[pl.ds(48, 16)])

            def lane(v, i):
                return jnp.reshape(lax.slice(v, (i,), (i + 1,)), ())

            tot = lane(vsum, 0)
            tot_b = lane(vsum, 1)
            nb = lane(vsum, 2)
            nm = lane(vsum, 3)
            inv_l = jnp.float32(1.0 / _L)

            def fast_fn(_):
                return jnp.where(nb >= 2.0, (tot + tot_b) * inv_l,
                                 tot * inv_l)

            def exact_fn(_):
                nch = 252
                t9 = jnp.float32(0.9)
                pltpu.sync_copy(x_hbm.at[pl.ds(img * 4096, 4096)], x_full)

                def fb_row(r, cc):
                    fbase = r * 64
                    rok = r < 63
                    for kk in range(4):
                        jc = iota + (kk * 16)
                        fa = fbase + kk * 16
                        ja = jnp.minimum(jc + fa, 4095)
                        jb = jnp.minimum(jc + (fa + 1), 4095)
                        jd = jnp.minimum(jc + (fa + 64), 4095)
                        je = jnp.minimum(jc + (fa + 65), 4095)
                        va = plsc.load_gather(x_full, [ja])
                        vb = plsc.load_gather(x_full, [jb])
                        vc = plsc.load_gather(x_full, [jd])
                        vd = plsc.load_gather(x_full, [je])
                        zz = va * w0 + vb * w1 + vc * w2 + vd * w3 + bb
                        ff = _sigmoid(zz)
                        ff = jnp.where((jc < 63) & rok, ff, 0.0)
                        f_full[pl.ds(fa, 16)] = ff
                    return cc

                lax.fori_loop(0, 64, fb_row, 0)

                def nz_body(ch, cc):
                    fc = f_full[pl.ds(ch * 16, 16)]
                    dd = fc + 1e-12
                    y = 1.0 / dd
                    y = y * (2.0 - dd * y)
                    y = y * (2.0 - dd * y)
                    nz_st[pl.ds(ch * 16, 16)] = fc * y
                    return cc

                lax.fori_loop(0, nch, nz_body, 0)

                def i_body(ich, tvec):
                    fv = f_full[pl.ds(ich * 16, 16)]
                    nv = nz_st[pl.ds(ich * 16, 16)]
                    deg_v = jnp.zeros((16,), jnp.float32)
                    agg_v = jnp.zeros((16,), jnp.float32)
                    for l in range(16):
                        nz_i = lane(nv, l)
                        f_i = lane(fv, l)
                        spl = jnp.full((16,), nz_i, jnp.float32)

                        def j_body(jch, c):
                            cnt, s = c
                            nzj = nz_st[pl.ds(jch * 16, 16)]
                            fj = f_full[pl.ds(jch * 16, 16)]
                            e = (spl * nzj) >= t9
                            return (cnt + jnp.where(e, 1.0, 0.0),
                                    s + jnp.where(e, fj, 0.0))

                        zz = jnp.zeros((16,), jnp.float32)
                        cnt, s = lax.fori_loop(0, nch, j_body, (zz, zz))
                        se = jnp.where(nz_i * nz_i >= t9, 1.0, 0.0)
                        deg = jnp.sum(cnt) - se
                        agg = jnp.sum(s) - se * f_i
                        deg_v = jnp.where(iota == l, deg, deg_v)
                        agg_v = jnp.where(iota == l, agg, agg_v)
                    dd = jnp.where(deg_v > 0, deg_v, 1.0)
                    y = 1.0 / dd
                    y = y * (2.0 - dd * y)
                    y = y * (2.0 - dd * y)
                    contrib = fv + jnp.where(deg_v > 0, agg_v * y, 0.0)
                    return tvec + contrib

                tvec = lax.fori_loop(0, nch, i_body,
                                     jnp.zeros((16,), jnp.float32))
                return jnp.sum(tvec) * inv_l

            res = lax.cond(nm > 0, exact_fn, fast_fn, 0)
            o_v[...] = jnp.where(iota == 0, res,
                                 jnp.where(iota == 1, nm, 0.0))
            pltpu.sync_copy(o_v, out_hbm.at[img])

    return k(xflat, wpack)


def kernel(x, W, b):
    xflat = x.reshape(-1)
    wpack = jnp.repeat(
        jnp.concatenate([W.reshape(-1), b.reshape(-1)]).astype(jnp.float32), 16
    )
    stats = _sc_graph_mean(xflat, wpack)
    return stats[:, :1]
